# baseline (speedup 1.0000x reference)
import functools

import jax
import jax.numpy as jnp
from jax import lax
from jax.experimental import pallas as pl
from jax.experimental.pallas import tpu as pltpu
from jax.experimental.pallas import tpu_sc as plsc

VOCAB = 100000
NSUB = 16
SLICE = 6272
TAIL = VOCAB - (NSUB - 1) * SLICE


def _sc_counts(token_counts, targets_flat):
    tgt_per_sub = targets_flat.shape[0] // NSUB

    mesh = plsc.VectorSubcoreMesh(core_axis_name="c", subcore_axis_name="s")

    @functools.partial(
        pl.kernel,
        out_type=jax.ShapeDtypeStruct((VOCAB,), jnp.float32),
        mesh=mesh,
        scratch_types=[
            pltpu.VMEM((TAIL,), jnp.float32),
            pltpu.VMEM((tgt_per_sub,), jnp.int32),
            pltpu.VMEM((tgt_per_sub,), jnp.float32),
            pltpu.VMEM_SHARED((VOCAB,), jnp.float32),
        ],
    )
    def k(tc_hbm, tgt_hbm, out_hbm, buf, idx_v, ones_v, shared):
        c = lax.axis_index("c")
        s = lax.axis_index("s")

        @pl.when(c == 0)
        def _():
            base = s * SLICE

            @pl.when(s < NSUB - 1)
            def _():
                pltpu.sync_copy(tc_hbm.at[pl.ds(base, SLICE)], shared.at[pl.ds(base, SLICE)])

            @pl.when(s == NSUB - 1)
            def _():
                pltpu.sync_copy(tc_hbm.at[pl.ds(base, TAIL)], buf)
                pltpu.sync_copy(buf, shared.at[pl.ds(base, TAIL)])

            pltpu.sync_copy(tgt_hbm.at[pl.ds(s * tgt_per_sub, tgt_per_sub)], idx_v)
            ones_v[...] = jnp.ones((tgt_per_sub,), jnp.float32)
            plsc.subcore_barrier()
            pltpu.sync_copy(ones_v, shared.at[idx_v], add=True)
            plsc.subcore_barrier()
            @pl.when(s < NSUB - 1)
            def _():
                pltpu.sync_copy(shared.at[pl.ds(base, SLICE)], out_hbm.at[pl.ds(base, SLICE)])

            @pl.when(s == NSUB - 1)
            def _():
                pltpu.sync_copy(shared.at[pl.ds(base, TAIL)], buf)
                pltpu.sync_copy(buf, out_hbm.at[pl.ds(base, TAIL)])

    return k(token_counts, targets_flat)


def _tc_pre(td, cwb, total_tokens, n_new):
    def body(tt_ref, td_ref, cwb_ref, thresh_ref, base_ref):
        total = jnp.maximum(tt_ref[0] + n_new, 1.0)
        thresh_ref[...] = 0.01 * jnp.maximum(td_ref[...], 1e-8) * total
        base_ref[...] = jnp.log(cwb_ref[...] * 0.99)

    return pl.pallas_call(
        body,
        in_specs=[
            pl.BlockSpec(memory_space=pltpu.SMEM),
            pl.BlockSpec((VOCAB,), lambda: (0,)),
            pl.BlockSpec((VOCAB,), lambda: (0,)),
        ],
        out_specs=[
            pl.BlockSpec((VOCAB,), lambda: (0,)),
            pl.BlockSpec((VOCAB,), lambda: (0,)),
        ],
        out_shape=[
            jax.ShapeDtypeStruct((VOCAB,), jnp.float32),
            jax.ShapeDtypeStruct((VOCAB,), jnp.float32),
        ],
    )(total_tokens, td, cwb)


_LOG_RATIO = 0.10536051565782628


def _tc_apply(logits, counts, thresh, base):
    b, s, vocab = logits.shape
    blk = 1
    grid = b // blk

    def body(tc_ref, th_ref, base_ref, x_ref, o_ref, lb_ref):
        @pl.when(pl.program_id(0) == 0)
        def _():
            lb_ref[...] = base_ref[...] + jnp.where(
                tc_ref[...] < th_ref[...], _LOG_RATIO, 0.0
            )

        o_ref[...] = x_ref[...] + lb_ref[...]

    return pl.pallas_call(
        body,
        grid=(grid,),
        in_specs=[
            pl.BlockSpec((vocab,), lambda i: (0,)),
            pl.BlockSpec((vocab,), lambda i: (0,)),
            pl.BlockSpec((vocab,), lambda i: (0,)),
            pl.BlockSpec((blk, s, vocab), lambda i: (i, 0, 0)),
        ],
        out_specs=pl.BlockSpec((blk, s, vocab), lambda i: (i, 0, 0)),
        out_shape=jax.ShapeDtypeStruct((b, s, vocab), jnp.float32),
        scratch_shapes=[pltpu.VMEM((vocab,), jnp.float32)],
    )(counts, thresh, base, logits)


def kernel(logits, targets, common_word_boost, target_dist, token_counts, total_tokens):
    counts = _sc_counts(token_counts, targets.reshape(-1).astype(jnp.int32))
    thresh, base = _tc_pre(
        target_dist, common_word_boost, total_tokens, float(targets.size)
    )
    return _tc_apply(logits, counts, thresh, base)


# speedup vs baseline: 1.0260x; 1.0260x over previous
import functools

import jax
import jax.numpy as jnp
from jax import lax
from jax.experimental import pallas as pl
from jax.experimental.pallas import tpu as pltpu
from jax.experimental.pallas import tpu_sc as plsc

VOCAB = 100000
NSUB = 16
SLICE = 6272
TAIL = VOCAB - (NSUB - 1) * SLICE


def _sc_counts(token_counts, targets_flat):
    tgt_per_sub = targets_flat.shape[0] // NSUB

    mesh = plsc.VectorSubcoreMesh(core_axis_name="c", subcore_axis_name="s")

    @functools.partial(
        pl.kernel,
        out_type=jax.ShapeDtypeStruct((VOCAB,), jnp.float32),
        mesh=mesh,
        scratch_types=[
            pltpu.VMEM((TAIL,), jnp.float32),
            pltpu.VMEM((tgt_per_sub,), jnp.int32),
            pltpu.VMEM((tgt_per_sub,), jnp.float32),
            pltpu.VMEM_SHARED((VOCAB,), jnp.float32),
        ],
    )
    def k(tc_hbm, tgt_hbm, out_hbm, buf, idx_v, ones_v, shared):
        c = lax.axis_index("c")
        s = lax.axis_index("s")

        @pl.when(c == 0)
        def _():
            base = s * SLICE

            @pl.when(s < NSUB - 1)
            def _():
                pltpu.sync_copy(tc_hbm.at[pl.ds(base, SLICE)], shared.at[pl.ds(base, SLICE)])

            @pl.when(s == NSUB - 1)
            def _():
                pltpu.sync_copy(tc_hbm.at[pl.ds(base, TAIL)], buf)
                pltpu.sync_copy(buf, shared.at[pl.ds(base, TAIL)])

            pltpu.sync_copy(tgt_hbm.at[pl.ds(s * tgt_per_sub, tgt_per_sub)], idx_v)
            ones_v[...] = jnp.ones((tgt_per_sub,), jnp.float32)
            plsc.subcore_barrier()
            pltpu.sync_copy(ones_v, shared.at[idx_v], add=True)
            plsc.subcore_barrier()
            @pl.when(s < NSUB - 1)
            def _():
                pltpu.sync_copy(shared.at[pl.ds(base, SLICE)], out_hbm.at[pl.ds(base, SLICE)])

            @pl.when(s == NSUB - 1)
            def _():
                pltpu.sync_copy(shared.at[pl.ds(base, TAIL)], buf)
                pltpu.sync_copy(buf, out_hbm.at[pl.ds(base, TAIL)])

    return k(token_counts, targets_flat)


def _tc_pre(td, cwb, total_tokens, n_new):
    def body(tt_ref, td_ref, cwb_ref, thresh_ref, base_ref):
        total = jnp.maximum(tt_ref[0] + n_new, 1.0)
        thresh_ref[...] = 0.01 * jnp.maximum(td_ref[...], 1e-8) * total
        base_ref[...] = jnp.log(cwb_ref[...] * 0.99)

    return pl.pallas_call(
        body,
        in_specs=[
            pl.BlockSpec(memory_space=pltpu.SMEM),
            pl.BlockSpec((VOCAB,), lambda: (0,)),
            pl.BlockSpec((VOCAB,), lambda: (0,)),
        ],
        out_specs=[
            pl.BlockSpec((VOCAB,), lambda: (0,)),
            pl.BlockSpec((VOCAB,), lambda: (0,)),
        ],
        out_shape=[
            jax.ShapeDtypeStruct((VOCAB,), jnp.float32),
            jax.ShapeDtypeStruct((VOCAB,), jnp.float32),
        ],
    )(total_tokens, td, cwb)


_LOG_RATIO = 0.10536051565782628


def _tc_apply(logits, counts, thresh, base):
    b, s, vocab = logits.shape
    blk = 2
    grid = b // blk

    def body(tc_ref, th_ref, base_ref, x_ref, o_ref, lb_ref):
        @pl.when(pl.program_id(0) == 0)
        def _():
            lb_ref[...] = base_ref[...] + jnp.where(
                tc_ref[...] < th_ref[...], _LOG_RATIO, 0.0
            )

        o_ref[...] = x_ref[...] + lb_ref[...]

    return pl.pallas_call(
        body,
        grid=(grid,),
        in_specs=[
            pl.BlockSpec((vocab,), lambda i: (0,)),
            pl.BlockSpec((vocab,), lambda i: (0,)),
            pl.BlockSpec((vocab,), lambda i: (0,)),
            pl.BlockSpec((blk, s, vocab), lambda i: (i, 0, 0)),
        ],
        out_specs=pl.BlockSpec((blk, s, vocab), lambda i: (i, 0, 0)),
        out_shape=jax.ShapeDtypeStruct((b, s, vocab), jnp.float32),
        scratch_shapes=[pltpu.VMEM((vocab,), jnp.float32)],
    )(counts, thresh, base, logits)


def kernel(logits, targets, common_word_boost, target_dist, token_counts, total_tokens):
    counts = _sc_counts(token_counts, targets.reshape(-1).astype(jnp.int32))
    thresh, base = _tc_pre(
        target_dist, common_word_boost, total_tokens, float(targets.size)
    )
    return _tc_apply(logits, counts, thresh, base)


# speedup vs baseline: 1.0432x; 1.0168x over previous
import functools

import jax
import jax.numpy as jnp
from jax import lax
from jax.experimental import pallas as pl
from jax.experimental.pallas import tpu as pltpu
from jax.experimental.pallas import tpu_sc as plsc

VOCAB = 100000
NSUB = 16
SLICE = 6272
TAIL = VOCAB - (NSUB - 1) * SLICE


def _sc_counts(token_counts, targets_flat):
    tgt_per_sub = targets_flat.shape[0] // NSUB

    mesh = plsc.VectorSubcoreMesh(core_axis_name="c", subcore_axis_name="s")

    @functools.partial(
        pl.kernel,
        out_type=jax.ShapeDtypeStruct((VOCAB,), jnp.float32),
        mesh=mesh,
        scratch_types=[
            pltpu.VMEM((TAIL,), jnp.float32),
            pltpu.VMEM((tgt_per_sub,), jnp.int32),
            pltpu.VMEM((tgt_per_sub,), jnp.float32),
            pltpu.VMEM_SHARED((VOCAB,), jnp.float32),
        ],
    )
    def k(tc_hbm, tgt_hbm, out_hbm, buf, idx_v, ones_v, shared):
        c = lax.axis_index("c")
        s = lax.axis_index("s")

        @pl.when(c == 0)
        def _():
            base = s * SLICE

            @pl.when(s < NSUB - 1)
            def _():
                pltpu.sync_copy(tc_hbm.at[pl.ds(base, SLICE)], shared.at[pl.ds(base, SLICE)])

            @pl.when(s == NSUB - 1)
            def _():
                pltpu.sync_copy(tc_hbm.at[pl.ds(base, TAIL)], buf)
                pltpu.sync_copy(buf, shared.at[pl.ds(base, TAIL)])

            pltpu.sync_copy(tgt_hbm.at[pl.ds(s * tgt_per_sub, tgt_per_sub)], idx_v)
            ones_v[...] = jnp.ones((tgt_per_sub,), jnp.float32)
            plsc.subcore_barrier()
            pltpu.sync_copy(ones_v, shared.at[idx_v], add=True)
            plsc.subcore_barrier()
            @pl.when(s < NSUB - 1)
            def _():
                pltpu.sync_copy(shared.at[pl.ds(base, SLICE)], out_hbm.at[pl.ds(base, SLICE)])

            @pl.when(s == NSUB - 1)
            def _():
                pltpu.sync_copy(shared.at[pl.ds(base, TAIL)], buf)
                pltpu.sync_copy(buf, out_hbm.at[pl.ds(base, TAIL)])

    return k(token_counts, targets_flat)


def _tc_pre(td, cwb, total_tokens, n_new):
    def body(tt_ref, td_ref, cwb_ref, thresh_ref, base_ref):
        total = jnp.maximum(tt_ref[0] + n_new, 1.0)
        thresh_ref[...] = 0.01 * jnp.maximum(td_ref[...], 1e-8) * total
        base_ref[...] = jnp.log(cwb_ref[...] * 0.99)

    return pl.pallas_call(
        body,
        in_specs=[
            pl.BlockSpec(memory_space=pltpu.SMEM),
            pl.BlockSpec((VOCAB,), lambda: (0,)),
            pl.BlockSpec((VOCAB,), lambda: (0,)),
        ],
        out_specs=[
            pl.BlockSpec((VOCAB,), lambda: (0,)),
            pl.BlockSpec((VOCAB,), lambda: (0,)),
        ],
        out_shape=[
            jax.ShapeDtypeStruct((VOCAB,), jnp.float32),
            jax.ShapeDtypeStruct((VOCAB,), jnp.float32),
        ],
    )(total_tokens, td, cwb)


_LOG_RATIO = 0.10536051565782628


def _tc_apply(logits, counts, thresh, base):
    b, s, vocab = logits.shape
    blk = 4
    grid = b // blk

    def body(tc_ref, th_ref, base_ref, x_ref, o_ref, lb_ref):
        @pl.when(pl.program_id(0) == 0)
        def _():
            lb_ref[...] = base_ref[...] + jnp.where(
                tc_ref[...] < th_ref[...], _LOG_RATIO, 0.0
            )

        o_ref[...] = x_ref[...] + lb_ref[...]

    return pl.pallas_call(
        body,
        grid=(grid,),
        in_specs=[
            pl.BlockSpec((vocab,), lambda i: (0,)),
            pl.BlockSpec((vocab,), lambda i: (0,)),
            pl.BlockSpec((vocab,), lambda i: (0,)),
            pl.BlockSpec((blk, s, vocab), lambda i: (i, 0, 0)),
        ],
        out_specs=pl.BlockSpec((blk, s, vocab), lambda i: (i, 0, 0)),
        out_shape=jax.ShapeDtypeStruct((b, s, vocab), jnp.float32),
        scratch_shapes=[pltpu.VMEM((vocab,), jnp.float32)],
    )(counts, thresh, base, logits)


def kernel(logits, targets, common_word_boost, target_dist, token_counts, total_tokens):
    counts = _sc_counts(token_counts, targets.reshape(-1).astype(jnp.int32))
    thresh, base = _tc_pre(
        target_dist, common_word_boost, total_tokens, float(targets.size)
    )
    return _tc_apply(logits, counts, thresh, base)


# speedup vs baseline: 1.0459x; 1.0026x over previous
import functools

import jax
import jax.numpy as jnp
from jax import lax
from jax.experimental import pallas as pl
from jax.experimental.pallas import tpu as pltpu
from jax.experimental.pallas import tpu_sc as plsc

VOCAB = 100000
NSUB = 16
H0 = 50176
CSLICE = H0 // NSUB
H1N = 15
CTAIL = VOCAB - H0 - H1N * CSLICE
DUMMY = H0


def _sc_counts(token_counts, targets_flat):
    tgt_per_sub = targets_flat.shape[0] // NSUB

    mesh = plsc.VectorSubcoreMesh(core_axis_name="c", subcore_axis_name="s")

    @functools.partial(
        pl.kernel,
        out_type=jax.ShapeDtypeStruct((VOCAB,), jnp.float32),
        mesh=mesh,
        scratch_types=[
            pltpu.VMEM((CSLICE,), jnp.float32),
            pltpu.VMEM((tgt_per_sub,), jnp.int32),
            pltpu.VMEM((tgt_per_sub,), jnp.int32),
            pltpu.VMEM((tgt_per_sub,), jnp.float32),
            pltpu.VMEM_SHARED((H0 + NSUB,), jnp.float32),
        ],
    )
    def k(tc_hbm, tgt_hbm, out_hbm, buf, idx_v, idxl_v, ones_v, shared):
        c = lax.axis_index("c")
        s = lax.axis_index("s")

        @pl.when(c == 0)
        def _():
            base = s * CSLICE
            pltpu.sync_copy(tc_hbm.at[pl.ds(base, CSLICE)], buf)
            pltpu.sync_copy(buf, shared.at[pl.ds(base, CSLICE)])

        @pl.when((c == 1) & (s < H1N))
        def _():
            gbase = H0 + s * CSLICE
            pltpu.sync_copy(tc_hbm.at[pl.ds(gbase, CSLICE)], buf)
            pltpu.sync_copy(buf, shared.at[pl.ds(s * CSLICE, CSLICE)])

        @pl.when((c == 1) & (s == H1N))
        def _():
            gbase = H0 + H1N * CSLICE
            pltpu.sync_copy(tc_hbm.at[pl.ds(gbase, CTAIL)], buf.at[pl.ds(0, CTAIL)])
            pltpu.sync_copy(buf.at[pl.ds(0, CTAIL)], shared.at[pl.ds(H1N * CSLICE, CTAIL)])

        pltpu.sync_copy(tgt_hbm.at[pl.ds(s * tgt_per_sub, tgt_per_sub)], idx_v)
        t = idx_v[...]
        start = c * H0
        upper = H0 + c * (VOCAB - H0)
        tl = t - start
        oob = (t < start) | (t >= upper)
        idxl_v[...] = jnp.where(oob, DUMMY, tl)
        ones_v[...] = jnp.ones((tgt_per_sub,), jnp.float32)
        plsc.subcore_barrier()
        pltpu.sync_copy(ones_v, shared.at[idxl_v], add=True)
        plsc.subcore_barrier()
        @pl.when(c == 0)
        def _():
            base = s * CSLICE
            pltpu.sync_copy(shared.at[pl.ds(base, CSLICE)], buf)
            pltpu.sync_copy(buf, out_hbm.at[pl.ds(base, CSLICE)])

        @pl.when((c == 1) & (s < H1N))
        def _():
            pltpu.sync_copy(shared.at[pl.ds(s * CSLICE, CSLICE)], buf)
            pltpu.sync_copy(buf, out_hbm.at[pl.ds(H0 + s * CSLICE, CSLICE)])

        @pl.when((c == 1) & (s == H1N))
        def _():
            pltpu.sync_copy(shared.at[pl.ds(H1N * CSLICE, CTAIL)], buf.at[pl.ds(0, CTAIL)])
            pltpu.sync_copy(buf.at[pl.ds(0, CTAIL)], out_hbm.at[pl.ds(H0 + H1N * CSLICE, CTAIL)])

    return k(token_counts, targets_flat)


def _tc_pre(td, cwb, total_tokens, n_new):
    def body(tt_ref, td_ref, cwb_ref, thresh_ref, base_ref):
        total = jnp.maximum(tt_ref[0] + n_new, 1.0)
        thresh_ref[...] = 0.01 * jnp.maximum(td_ref[...], 1e-8) * total
        base_ref[...] = jnp.log(cwb_ref[...] * 0.99)

    return pl.pallas_call(
        body,
        in_specs=[
            pl.BlockSpec(memory_space=pltpu.SMEM),
            pl.BlockSpec((VOCAB,), lambda: (0,)),
            pl.BlockSpec((VOCAB,), lambda: (0,)),
        ],
        out_specs=[
            pl.BlockSpec((VOCAB,), lambda: (0,)),
            pl.BlockSpec((VOCAB,), lambda: (0,)),
        ],
        out_shape=[
            jax.ShapeDtypeStruct((VOCAB,), jnp.float32),
            jax.ShapeDtypeStruct((VOCAB,), jnp.float32),
        ],
    )(total_tokens, td, cwb)


_LOG_RATIO = 0.10536051565782628


def _tc_apply(logits, counts, thresh, base):
    b, s, vocab = logits.shape
    blk = 4
    grid = b // blk

    def body(tc_ref, th_ref, base_ref, x_ref, o_ref, lb_ref):
        @pl.when(pl.program_id(0) == 0)
        def _():
            lb_ref[...] = base_ref[...] + jnp.where(
                tc_ref[...] < th_ref[...], _LOG_RATIO, 0.0
            )

        o_ref[...] = x_ref[...] + lb_ref[...]

    return pl.pallas_call(
        body,
        grid=(grid,),
        in_specs=[
            pl.BlockSpec((vocab,), lambda i: (0,)),
            pl.BlockSpec((vocab,), lambda i: (0,)),
            pl.BlockSpec((vocab,), lambda i: (0,)),
            pl.BlockSpec((blk, s, vocab), lambda i: (i, 0, 0)),
        ],
        out_specs=pl.BlockSpec((blk, s, vocab), lambda i: (i, 0, 0)),
        out_shape=jax.ShapeDtypeStruct((b, s, vocab), jnp.float32),
        scratch_shapes=[pltpu.VMEM((vocab,), jnp.float32)],
    )(counts, thresh, base, logits)


def kernel(logits, targets, common_word_boost, target_dist, token_counts, total_tokens):
    counts = _sc_counts(token_counts, targets.reshape(-1).astype(jnp.int32))
    thresh, base = _tc_pre(
        target_dist, common_word_boost, total_tokens, float(targets.size)
    )
    return _tc_apply(logits, counts, thresh, base)
